# baseline (speedup 1.0000x reference)
import jax
import jax.numpy as jnp
from jax import lax
from jax.experimental import pallas as pl
from jax.experimental.pallas import tpu as pltpu
from jax.experimental.pallas import tpu_sc as plsc

B, N, D = 256, 128, 768
R = N + 1
RB = 17
LB = D // 128
BW = LB * 8 * 128
NC, NS = 2, 16
NW = NC * NS
BPW = B // NW
C = 8
CPB = RB
NCH = BPW * CPB
NIDX = NCH + 2
LANES = 16
VECS = D // LANES
NA, NDEG = 119 + 1, 256 + 1
NT = NA + 2 * NDEG + 1


def _sc_body(i1_hbm, i2_hbm, i3_hbm, tab_hbm, out_hbm,
             idx1, idx2, idx3, rows1, rows2, rows3, obuf, tab_sh,
             sem_g0, sem_g1, sem_w0, sem_w1):
    cid = lax.axis_index("c")
    sid = lax.axis_index("s")
    wid = sid * NC + cid
    base_b = wid * BPW
    sem_g = (sem_g0, sem_g1)
    sem_w = (sem_w0, sem_w1)

    HALF = 320

    @pl.when(sid == 0)
    def _():
        pltpu.sync_copy(tab_hbm.at[pl.ds(0, HALF)], tab_sh.at[pl.ds(0, HALF)])

    @pl.when(sid == 1)
    def _():
        pltpu.sync_copy(tab_hbm.at[pl.ds(HALF, NT - HALF)],
                        tab_sh.at[pl.ds(HALF, NT - HALF)])

    pltpu.sync_copy(i1_hbm.at[wid], idx1)
    pltpu.sync_copy(i2_hbm.at[wid], idx2)
    pltpu.sync_copy(i3_hbm.at[wid], idx3)

    plsc.subcore_barrier()

    def issue_gathers(k, p):
        d1 = pltpu.async_copy(tab_sh.at[idx1.at[k]], rows1.at[p], sem_g[p])
        d2 = pltpu.async_copy(tab_sh.at[idx2.at[k]], rows2.at[p], sem_g[p])
        d3 = pltpu.async_copy(tab_sh.at[idx3.at[k]], rows3.at[p], sem_g[p])
        return (d1, d2, d3)

    def drain_gathers(p):
        pltpu.make_async_copy(tab_hbm.at[pl.ds(0, C)], rows1.at[p],
                              sem_g[p]).wait()
        pltpu.make_async_copy(tab_hbm.at[pl.ds(0, C)], rows2.at[p],
                              sem_g[p]).wait()
        pltpu.make_async_copy(tab_hbm.at[pl.ds(0, C)], rows3.at[p],
                              sem_g[p]).wait()

    issue_gathers(0, 0)
    issue_gathers(1, 1)

    def batch_body(bi, carry):
        row0 = (base_b + bi) * RB
        k0 = bi * CPB
        gat = [None, None]
        wr = [None] * CPB
        for j in range(CPB):
            p = j % 2
            q = j % 3
            if j < 2:
                drain_gathers(p)
            else:
                for d in gat[p]:
                    d.wait()
            if j >= 3:
                wr[j - 3].wait()

            @plsc.parallel_loop(0, C * VECS, unroll=16)
            def add_body(i):
                r = i // VECS
                v = i % VECS
                off = (v // 8) * 1024 + r * 128 + (v % 8) * LANES
                x = (rows1[p, r, pl.ds(v * LANES, LANES)]
                     + rows2[p, r, pl.ds(v * LANES, LANES)]
                     + rows3[p, r, pl.ds(v * LANES, LANES)])
                obuf[q, 0, pl.ds(off, LANES)] = x

            if j <= CPB - 3:
                gat[p] = issue_gathers(k0 + j + 2, p)
            if j == CPB - 1:
                issue_gathers(k0 + CPB, 0)
                issue_gathers(k0 + CPB + 1, 1)
            wr[j] = pltpu.async_copy(
                obuf.at[q], out_hbm.at[pl.ds(row0 + j, 1)], sem_w[p])
        wr[CPB - 3].wait()
        wr[CPB - 2].wait()
        wr[CPB - 1].wait()
        return carry

    lax.fori_loop(0, BPW, batch_body, 0)

    drain_gathers(0)
    drain_gathers(1)


@jax.jit
def _sc_embed(i1, i2, i3, tab):
    mesh = plsc.VectorSubcoreMesh(core_axis_name="c", subcore_axis_name="s",
                                  num_cores=NC, num_subcores=NS)
    return pl.kernel(
        _sc_body,
        out_type=jax.ShapeDtypeStruct((B * RB, BW), jnp.float32),
        mesh=mesh,
        scratch_types=[
            pltpu.VMEM((NIDX, C), jnp.int32),
            pltpu.VMEM((NIDX, C), jnp.int32),
            pltpu.VMEM((NIDX, C), jnp.int32),
            pltpu.VMEM((2, C, D), jnp.float32),
            pltpu.VMEM((2, C, D), jnp.float32),
            pltpu.VMEM((2, C, D), jnp.float32),
            pltpu.VMEM((3, 1, BW), jnp.float32),
            pltpu.VMEM_SHARED((NT, D), jnp.float32),
            pltpu.SemaphoreType.DMA,
            pltpu.SemaphoreType.DMA,
            pltpu.SemaphoreType.DMA,
            pltpu.SemaphoreType.DMA,
        ],
        compiler_params=pltpu.CompilerParams(use_tc_tiling_on_sc=False),
    )(i1, i2, i3, tab)


def _prep_indices(atom_types, in_degrees, out_degrees):
    at = atom_types.astype(jnp.int32)
    ind = in_degrees.astype(jnp.int32) + NA
    od = out_degrees.astype(jnp.int32) + NA + NDEG
    vcol = jnp.full((B, 1), NT - 1, jnp.int32)
    zcol = jnp.zeros((B, 1), jnp.int32)

    def prep(first_col, body):
        x = jnp.concatenate([first_col, body], axis=1)
        x = jnp.pad(x, ((0, 0), (0, CPB * C - R)))
        x = x.reshape(NW, NCH, C)
        return jnp.pad(x, ((0, 0), (0, NIDX - NCH), (0, 0)))

    return prep(vcol, at), prep(zcol, ind), prep(zcol, od)


def kernel(atom_types, in_degrees, out_degrees, atom_table, in_table,
           out_table, vnode_table):
    i1, i2, i3 = _prep_indices(atom_types, in_degrees, out_degrees)
    tab = jnp.concatenate([atom_table, in_table, out_table, vnode_table],
                          axis=0)
    out2 = _sc_embed(i1, i2, i3, tab)
    out5 = out2.reshape(B, RB, LB, 8, 128)
    out = out5.transpose(0, 1, 3, 2, 4).reshape(B, RB * 8, D)
    return out[:, :R, :]


# speedup vs baseline: 1.4039x; 1.4039x over previous
import jax
import jax.numpy as jnp
from jax import lax
from jax.experimental import pallas as pl
from jax.experimental.pallas import tpu as pltpu
from jax.experimental.pallas import tpu_sc as plsc

B, N, D = 256, 128, 768
R = N + 1
RB = 17
LB = D // 128
BW = LB * 8 * 128
NC, NS = 2, 16
NW = NC * NS
BPW = B // NW
C = 8
CPB = RB
NCH = BPW * CPB
NIDX = NCH + 2
LANES = 16
VECS = D // LANES
NA, NDEG = 119 + 1, 256 + 1
NT = NA + 2 * NDEG + 1


def _sc_body(i1_hbm, i2_hbm, i3_hbm, tab_hbm, out_hbm,
             idx1, idx2, idx3, rows1, rows2, rows3, obuf, tab_sh,
             sem_g0, sem_g1, sem_w0, sem_w1):
    cid = lax.axis_index("c")
    sid = lax.axis_index("s")
    wid = sid * NC + cid
    base_b = wid * BPW
    sem_g = (sem_g0, sem_g1)
    sem_w = (sem_w0, sem_w1)

    HALF = 320

    @pl.when(sid == 0)
    def _():
        pltpu.sync_copy(tab_hbm.at[pl.ds(0, HALF)], tab_sh.at[pl.ds(0, HALF)])

    @pl.when(sid == 1)
    def _():
        pltpu.sync_copy(tab_hbm.at[pl.ds(HALF, NT - HALF)],
                        tab_sh.at[pl.ds(HALF, NT - HALF)])

    pltpu.sync_copy(i1_hbm.at[wid], idx1)
    pltpu.sync_copy(i2_hbm.at[wid], idx2)
    pltpu.sync_copy(i3_hbm.at[wid], idx3)

    plsc.subcore_barrier()

    def issue_gathers(k, p):
        d1 = pltpu.async_copy(tab_sh.at[idx1.at[k]], rows1.at[p], sem_g[p])
        d2 = pltpu.async_copy(tab_sh.at[idx2.at[k]], rows2.at[p], sem_g[p])
        d3 = pltpu.async_copy(tab_sh.at[idx3.at[k]], rows3.at[p], sem_g[p])
        return (d1, d2, d3)

    def drain_gathers(p):
        pltpu.make_async_copy(tab_hbm.at[pl.ds(0, C)], rows1.at[p],
                              sem_g[p]).wait()
        pltpu.make_async_copy(tab_hbm.at[pl.ds(0, C)], rows2.at[p],
                              sem_g[p]).wait()
        pltpu.make_async_copy(tab_hbm.at[pl.ds(0, C)], rows3.at[p],
                              sem_g[p]).wait()

    issue_gathers(0, 0)
    issue_gathers(1, 1)

    def batch_body(bi, carry):
        row0 = (base_b + bi) * RB
        k0 = bi * CPB
        gat = [None, None]
        wr = [None] * CPB
        for j in range(CPB):
            p = j % 2
            q = j % 3
            if j < 2:
                drain_gathers(p)
            else:
                for d in gat[p]:
                    d.wait()
            if j >= 3:
                wr[j - 3].wait()

            @plsc.parallel_loop(0, C * VECS, unroll=8)
            def add_body(i):
                r = i // VECS
                v = i % VECS
                off = (v // 8) * 1024 + r * 128 + (v % 8) * LANES
                x = (rows1[p, r, pl.ds(v * LANES, LANES)]
                     + rows2[p, r, pl.ds(v * LANES, LANES)]
                     + rows3[p, r, pl.ds(v * LANES, LANES)])
                obuf[q, 0, pl.ds(off, LANES)] = x

            if j <= CPB - 3:
                gat[p] = issue_gathers(k0 + j + 2, p)
            if j == CPB - 1:
                issue_gathers(k0 + CPB, 0)
                issue_gathers(k0 + CPB + 1, 1)
            wr[j] = pltpu.async_copy(
                obuf.at[q], out_hbm.at[pl.ds(row0 + j, 1)], sem_w[p])
        wr[CPB - 3].wait()
        wr[CPB - 2].wait()
        wr[CPB - 1].wait()
        return carry

    lax.fori_loop(0, BPW, batch_body, 0)

    drain_gathers(0)
    drain_gathers(1)


@jax.jit
def _sc_embed(i1, i2, i3, tab):
    mesh = plsc.VectorSubcoreMesh(core_axis_name="c", subcore_axis_name="s",
                                  num_cores=NC, num_subcores=NS)
    return pl.kernel(
        _sc_body,
        out_type=jax.ShapeDtypeStruct((B * RB, BW), jnp.float32),
        mesh=mesh,
        scratch_types=[
            pltpu.VMEM((NIDX, C), jnp.int32),
            pltpu.VMEM((NIDX, C), jnp.int32),
            pltpu.VMEM((NIDX, C), jnp.int32),
            pltpu.VMEM((2, C, D), jnp.float32),
            pltpu.VMEM((2, C, D), jnp.float32),
            pltpu.VMEM((2, C, D), jnp.float32),
            pltpu.VMEM((3, 1, BW), jnp.float32),
            pltpu.VMEM_SHARED((NT, D), jnp.float32),
            pltpu.SemaphoreType.DMA,
            pltpu.SemaphoreType.DMA,
            pltpu.SemaphoreType.DMA,
            pltpu.SemaphoreType.DMA,
        ],
        compiler_params=pltpu.CompilerParams(use_tc_tiling_on_sc=False),
    )(i1, i2, i3, tab)


def _prep_indices(atom_types, in_degrees, out_degrees):
    at = atom_types.astype(jnp.int32)
    ind = in_degrees.astype(jnp.int32) + NA
    od = out_degrees.astype(jnp.int32) + NA + NDEG
    vcol = jnp.full((B, 1), NT - 1, jnp.int32)
    zcol = jnp.zeros((B, 1), jnp.int32)

    def prep(first_col, body):
        x = jnp.concatenate([first_col, body], axis=1)
        x = jnp.pad(x, ((0, 0), (0, CPB * C - R)))
        x = x.reshape(NW, NCH, C)
        return jnp.pad(x, ((0, 0), (0, NIDX - NCH), (0, 0)))

    return prep(vcol, at), prep(zcol, ind), prep(zcol, od)


def kernel(atom_types, in_degrees, out_degrees, atom_table, in_table,
           out_table, vnode_table):
    i1, i2, i3 = _prep_indices(atom_types, in_degrees, out_degrees)
    tab = jnp.concatenate([atom_table, in_table, out_table, vnode_table],
                          axis=0)
    out2 = _sc_embed(i1, i2, i3, tab)
    out5 = out2.reshape(B, RB, LB, 8, 128)
    out = out5.transpose(0, 1, 3, 2, 4).reshape(B, RB * 8, D)
    return out[:, :R, :]
